# baseline (speedup 1.0000x reference)
import functools

import jax
import jax.numpy as jnp
from jax import lax
from jax.experimental import pallas as pl
from jax.experimental.pallas import tpu as pltpu
from jax.experimental.pallas import tpu_sc as plsc

_HI = jax.lax.Precision.HIGHEST

_P_TC = 64


def _bank_kernel(x_ref, y_ref, b2_ref, k1_ref, k2_ref, k3_ref,
                 ym_ref, xfeat_ref):
    x = x_ref[...]
    x_feat = jnp.sum(x, axis=1) * 0.125
    xfeat_ref[...] = x_feat
    keys_un = jnp.dot(x_feat, b2_ref[...],
                      preferred_element_type=jnp.float32, precision=_HI)
    ss = jnp.sum(keys_un * keys_un, axis=-1, keepdims=True)
    nrm = jnp.maximum(jnp.sqrt(ss), 1e-12)
    keys = keys_un / nrm
    k1_ref[...] = keys
    k2_ref[...] = keys
    k3_ref[...] = keys
    y = y_ref[...]
    ym_ref[...] = jnp.sum(y, axis=1) * 0.125


def _make_ymean_sc(P0, P, N, M):
    NC = 2
    mesh = plsc.VectorSubcoreMesh(core_axis_name="c", subcore_axis_name="s")

    @functools.partial(
        pl.kernel,
        mesh=mesh,
        out_type=jax.ShapeDtypeStruct((P - P0, M), jnp.float32),
        scratch_types=[
            pltpu.VMEM((N, M), jnp.float32),
            pltpu.VMEM((M,), jnp.float32),
        ],
    )
    def ymean_sc(y_hbm, out_hbm, rows_v, acc_v):
        wid = lax.axis_index("s") * NC + lax.axis_index("c")
        pltpu.sync_copy(y_hbm.at[P0 + wid], rows_v)

        def do_chunk(i, _):
            s = i * 16
            v = rows_v[0, pl.ds(s, 16)]
            for c in range(1, N):
                v = v + rows_v[c, pl.ds(s, 16)]
            acc_v[pl.ds(s, 16)] = v * 0.125
            return 0

        lax.fori_loop(0, M // 16, do_chunk, 0)
        pltpu.sync_copy(acc_v, out_hbm.at[wid])

    return ymean_sc


def kernel(all_x, all_y, w_ext, b_ext, w_cp, b_cp, W_enc):
    M, T, N = all_x.shape
    P = all_y.shape[1]
    BINS, D = W_enc.shape

    xt = jnp.transpose(all_x, (0, 2, 1))
    yt = jnp.transpose(all_y, (1, 2, 0))

    B2 = jnp.repeat(W_enc, T // BINS, axis=0) / (T // BINS)

    ym_tail = _make_ymean_sc(_P_TC, P, N, M)(yt)

    BM = 256
    grid = (M // BM,)
    k1, k2, k3, ym_head, x_feat = pl.pallas_call(
        _bank_kernel,
        grid=grid,
        in_specs=[
            pl.BlockSpec((BM, N, T), lambda i: (i, 0, 0)),
            pl.BlockSpec((_P_TC, N, BM), lambda i: (0, 0, i)),
            pl.BlockSpec((T, D), lambda i: (0, 0)),
        ],
        out_specs=[
            pl.BlockSpec((BM, D), lambda i: (i, 0)),
            pl.BlockSpec((BM, D), lambda i: (i, 0)),
            pl.BlockSpec((BM, D), lambda i: (i, 0)),
            pl.BlockSpec((_P_TC, BM), lambda i: (0, i)),
            pl.BlockSpec((BM, T), lambda i: (i, 0)),
        ],
        out_shape=[
            jax.ShapeDtypeStruct((M, D), jnp.float32),
            jax.ShapeDtypeStruct((M, D), jnp.float32),
            jax.ShapeDtypeStruct((M, D), jnp.float32),
            jax.ShapeDtypeStruct((_P_TC, M), jnp.float32),
            jax.ShapeDtypeStruct((M, T), jnp.float32),
        ],
    )(xt, yt, B2)
    ym = jnp.concatenate([ym_head, ym_tail], axis=0).T

    extreme_probs = jax.nn.sigmoid(x_feat @ w_ext + b_ext)
    near_end_scores = jax.nn.sigmoid(x_feat[:, -64:] @ w_cp + b_cp)
    labels = jnp.zeros((M,), dtype=jnp.int32)
    labels = jnp.where(extreme_probs > 0.5, jnp.int32(1), labels)
    labels = jnp.where(near_end_scores > 0.5, jnp.int32(2), labels)
    return (k1, k2, k3, ym, labels)


# speedup vs baseline: 1.3059x; 1.3059x over previous
import jax
import jax.numpy as jnp
from jax.experimental import pallas as pl

_HI = jax.lax.Precision.HIGHEST


def _bank_kernel(x_ref, y_ref, b2_ref, k1_ref, k2_ref, k3_ref,
                 ym_ref, xfeat_ref):
    x = x_ref[...]
    x_feat = jnp.sum(x, axis=1) * 0.125
    xfeat_ref[...] = x_feat
    keys_un = jnp.dot(x_feat, b2_ref[...],
                      preferred_element_type=jnp.float32, precision=_HI)
    ss = jnp.sum(keys_un * keys_un, axis=-1, keepdims=True)
    nrm = jnp.maximum(jnp.sqrt(ss), 1e-12)
    keys = keys_un / nrm
    k1_ref[...] = keys
    k2_ref[...] = keys
    k3_ref[...] = keys
    y = y_ref[...]
    ym_ref[...] = jnp.sum(y, axis=1) * 0.125


def kernel(all_x, all_y, w_ext, b_ext, w_cp, b_cp, W_enc):
    M, T, N = all_x.shape
    P = all_y.shape[1]
    BINS, D = W_enc.shape

    xt = jnp.transpose(all_x, (0, 2, 1))
    yt = jnp.transpose(all_y, (1, 2, 0))

    B2 = jnp.repeat(W_enc, T // BINS, axis=0) / (T // BINS)

    BM = 512
    grid = (M // BM,)
    k1, k2, k3, ym_t, x_feat = pl.pallas_call(
        _bank_kernel,
        grid=grid,
        in_specs=[
            pl.BlockSpec((BM, N, T), lambda i: (i, 0, 0)),
            pl.BlockSpec((P, N, BM), lambda i: (0, 0, i)),
            pl.BlockSpec((T, D), lambda i: (0, 0)),
        ],
        out_specs=[
            pl.BlockSpec((BM, D), lambda i: (i, 0)),
            pl.BlockSpec((BM, D), lambda i: (i, 0)),
            pl.BlockSpec((BM, D), lambda i: (i, 0)),
            pl.BlockSpec((P, BM), lambda i: (0, i)),
            pl.BlockSpec((BM, T), lambda i: (i, 0)),
        ],
        out_shape=[
            jax.ShapeDtypeStruct((M, D), jnp.float32),
            jax.ShapeDtypeStruct((M, D), jnp.float32),
            jax.ShapeDtypeStruct((M, D), jnp.float32),
            jax.ShapeDtypeStruct((P, M), jnp.float32),
            jax.ShapeDtypeStruct((M, T), jnp.float32),
        ],
    )(xt, yt, B2)
    ym = ym_t.T

    extreme_probs = jax.nn.sigmoid(x_feat @ w_ext + b_ext)
    near_end_scores = jax.nn.sigmoid(x_feat[:, -64:] @ w_cp + b_cp)
    labels = jnp.zeros((M,), dtype=jnp.int32)
    labels = jnp.where(extreme_probs > 0.5, jnp.int32(1), labels)
    labels = jnp.where(near_end_scores > 0.5, jnp.int32(2), labels)
    return (k1, k2, k3, ym, labels)


# speedup vs baseline: 1.3589x; 1.0406x over previous
import jax
import jax.numpy as jnp
from jax.experimental import pallas as pl

_HI = jax.lax.Precision.HIGHEST


def _bank_kernel(x_ref, y_ref, b2_ref, k1_ref, k2_ref, k3_ref,
                 ym_ref, xfeat_ref):
    x = x_ref[...]
    x_feat = jnp.sum(x, axis=1) * 0.125
    xfeat_ref[...] = x_feat
    keys_un = jnp.dot(x_feat, b2_ref[...],
                      preferred_element_type=jnp.float32, precision=_HI)
    ss = jnp.sum(keys_un * keys_un, axis=-1, keepdims=True)
    nrm = jnp.maximum(jnp.sqrt(ss), 1e-12)
    keys = keys_un / nrm
    k1_ref[...] = keys
    k2_ref[...] = keys
    k3_ref[...] = keys
    y = y_ref[...]
    ym_ref[...] = jnp.sum(y, axis=1) * 0.125


def kernel(all_x, all_y, w_ext, b_ext, w_cp, b_cp, W_enc):
    M, T, N = all_x.shape
    P = all_y.shape[1]
    BINS, D = W_enc.shape

    xt = jnp.transpose(all_x, (0, 2, 1))
    yt = jnp.transpose(all_y, (1, 2, 0))

    B2 = jnp.repeat(W_enc, T // BINS, axis=0) / (T // BINS)

    BM = 1024
    grid = (M // BM,)
    k1, k2, k3, ym_t, x_feat = pl.pallas_call(
        _bank_kernel,
        grid=grid,
        in_specs=[
            pl.BlockSpec((BM, N, T), lambda i: (i, 0, 0)),
            pl.BlockSpec((P, N, BM), lambda i: (0, 0, i)),
            pl.BlockSpec((T, D), lambda i: (0, 0)),
        ],
        out_specs=[
            pl.BlockSpec((BM, D), lambda i: (i, 0)),
            pl.BlockSpec((BM, D), lambda i: (i, 0)),
            pl.BlockSpec((BM, D), lambda i: (i, 0)),
            pl.BlockSpec((P, BM), lambda i: (0, i)),
            pl.BlockSpec((BM, T), lambda i: (i, 0)),
        ],
        out_shape=[
            jax.ShapeDtypeStruct((M, D), jnp.float32),
            jax.ShapeDtypeStruct((M, D), jnp.float32),
            jax.ShapeDtypeStruct((M, D), jnp.float32),
            jax.ShapeDtypeStruct((P, M), jnp.float32),
            jax.ShapeDtypeStruct((M, T), jnp.float32),
        ],
    )(xt, yt, B2)
    ym = ym_t.T

    extreme_probs = jax.nn.sigmoid(x_feat @ w_ext + b_ext)
    near_end_scores = jax.nn.sigmoid(x_feat[:, -64:] @ w_cp + b_cp)
    labels = jnp.zeros((M,), dtype=jnp.int32)
    labels = jnp.where(extreme_probs > 0.5, jnp.int32(1), labels)
    labels = jnp.where(near_end_scores > 0.5, jnp.int32(2), labels)
    return (k1, k2, k3, ym, labels)
